# Initial kernel scaffold; baseline (speedup 1.0000x reference)
import jax
import jax.numpy as jnp
from jax.experimental import pallas as pl


def kernel(inputs, hidden_states, edge_index, edge_attr, W1, b1, W_ih, b_ih, W_hh, b_hh, Wl, Wr, We, att, gnn_bias, W2, b2):
    raise NotImplementedError("write your pallas kernel here")


# speedup vs baseline: 2.5539x; 2.5539x over previous
import functools

import jax
import jax.numpy as jnp
from jax.experimental import pallas as pl

N = 10000
E = 320000
D_IN = 128
H = 64
HH = 2 * H
N_ACT = 10
EDGE_DIM = 5

ROW_BLK = 1000


def _dense_pre_body(inp_ref, hid_ref, W1_ref, b1_ref, WihT_ref, bih_ref,
                    WhhT_ref, bhh_ref, Wl_ref, Wr_ref,
                    hs_ref, xl_ref, xr_ref):
    x = jnp.dot(inp_ref[...], W1_ref[...], preferred_element_type=jnp.float32)
    x = jax.nn.relu(x + b1_ref[...])
    h = hid_ref[...]
    gi = jnp.dot(x, WihT_ref[...], preferred_element_type=jnp.float32) + bih_ref[...]
    gh = jnp.dot(h, WhhT_ref[...], preferred_element_type=jnp.float32) + bhh_ref[...]
    r = jax.nn.sigmoid(gi[:, :HH] + gh[:, :HH])
    z = jax.nn.sigmoid(gi[:, HH:2 * HH] + gh[:, HH:2 * HH])
    n = jnp.tanh(gi[:, 2 * HH:] + r * gh[:, 2 * HH:])
    h_s = (1.0 - z) * n + z * h
    hs_ref[...] = h_s
    xl_ref[...] = jnp.dot(h_s, Wl_ref[...], preferred_element_type=jnp.float32)
    xr_ref[...] = jnp.dot(h_s, Wr_ref[...], preferred_element_type=jnp.float32)


def _dense_pre(inputs, hidden, W1, b1, W_ihT, b_ih, W_hhT, b_hh, Wl, Wr):
    nblk = N // ROW_BLK
    full = lambda shape: pl.BlockSpec(shape, lambda i: (0,) * len(shape))
    row = lambda c: pl.BlockSpec((ROW_BLK, c), lambda i: (i, 0))
    return pl.pallas_call(
        _dense_pre_body,
        grid=(nblk,),
        in_specs=[
            row(D_IN), row(HH),
            full((D_IN, H)), full((1, H)),
            full((H, 3 * HH)), full((1, 3 * HH)),
            full((HH, 3 * HH)), full((1, 3 * HH)),
            full((HH, H)), full((HH, H)),
        ],
        out_specs=[row(HH), row(H), row(H)],
        out_shape=[
            jax.ShapeDtypeStruct((N, HH), jnp.float32),
            jax.ShapeDtypeStruct((N, H), jnp.float32),
            jax.ShapeDtypeStruct((N, H), jnp.float32),
        ],
    )(inputs, hidden, W1, b1, W_ihT, b_ih, W_hhT, b_hh, Wl, Wr)


def kernel(inputs, hidden_states, edge_index, edge_attr,
           W1, b1, W_ih, b_ih, W_hh, b_hh,
           Wl, Wr, We, att, gnn_bias, W2, b2):
    h_s, xl, xr = _dense_pre(
        inputs, hidden_states.reshape(-1, HH),
        W1, b1.reshape(1, H),
        W_ih.T, b_ih.reshape(1, 3 * HH),
        W_hh.T, b_hh.reshape(1, 3 * HH),
        Wl, Wr)

    src = edge_index[0]
    dst = edge_index[1]
    e = edge_attr @ We
    m = xl[src] + xr[dst] + e
    logits = jax.nn.leaky_relu(m, 0.2) @ att
    ex = jnp.exp(logits)
    denom = jax.ops.segment_sum(ex, dst, num_segments=N)
    num = jax.ops.segment_sum(ex[:, None] * xl[src], dst, num_segments=N)
    out = num / (denom[:, None] + 1e-16)
    h = jax.nn.relu(out + gnn_bias)
    q = h @ W2 + b2
    return (q, h_s)


# speedup vs baseline: 4.0685x; 1.5930x over previous
import functools

import jax
import jax.numpy as jnp
from jax import lax
from jax.experimental import pallas as pl
from jax.experimental.pallas import tpu as pltpu
from jax.experimental.pallas import tpu_sc as plsc

N = 10000
E = 320000
D_IN = 128
H = 64
HH = 2 * H
N_ACT = 10
EDGE_DIM = 5

ROW_BLK = 1000
EDGE_BLK = 8000

NC = 2
NS = 16
NW = NC * NS
CHUNK = 128
NCHUNK = E // CHUNK
LOOPS = -(-NCHUNK // NW)


def _dense_pre_body(inp_ref, hid_ref, W1_ref, b1_ref, WihT_ref, bih_ref,
                    WhhT_ref, bhh_ref, Wl_ref, Wr_ref,
                    hs_ref, xl128_ref, xr128_ref):
    x = jnp.dot(inp_ref[...], W1_ref[...], preferred_element_type=jnp.float32)
    x = jax.nn.relu(x + b1_ref[...])
    h = hid_ref[...]
    gi = jnp.dot(x, WihT_ref[...], preferred_element_type=jnp.float32) + bih_ref[...]
    gh = jnp.dot(h, WhhT_ref[...], preferred_element_type=jnp.float32) + bhh_ref[...]
    r = jax.nn.sigmoid(gi[:, :HH] + gh[:, :HH])
    z = jax.nn.sigmoid(gi[:, HH:2 * HH] + gh[:, HH:2 * HH])
    n = jnp.tanh(gi[:, 2 * HH:] + r * gh[:, 2 * HH:])
    h_s = (1.0 - z) * n + z * h
    hs_ref[...] = h_s
    xl = jnp.dot(h_s, Wl_ref[...], preferred_element_type=jnp.float32)
    xr = jnp.dot(h_s, Wr_ref[...], preferred_element_type=jnp.float32)
    zero = jnp.zeros_like(xl)
    xl128_ref[...] = jnp.concatenate([xl, zero], axis=1)
    xr128_ref[...] = jnp.concatenate([zero, xr], axis=1)


def _dense_pre(inputs, hidden, W1, b1, W_ihT, b_ih, W_hhT, b_hh, Wl, Wr):
    nblk = N // ROW_BLK
    full = lambda shape: pl.BlockSpec(shape, lambda i: (0,) * len(shape))
    row = lambda c: pl.BlockSpec((ROW_BLK, c), lambda i: (i, 0))
    return pl.pallas_call(
        _dense_pre_body,
        grid=(nblk,),
        in_specs=[
            row(D_IN), row(HH),
            full((D_IN, H)), full((1, H)),
            full((H, 3 * HH)), full((1, 3 * HH)),
            full((HH, 3 * HH)), full((1, 3 * HH)),
            full((HH, H)), full((HH, H)),
        ],
        out_specs=[row(HH), row(2 * H), row(2 * H)],
        out_shape=[
            jax.ShapeDtypeStruct((N, HH), jnp.float32),
            jax.ShapeDtypeStruct((N, 2 * H), jnp.float32),
            jax.ShapeDtypeStruct((N, 2 * H), jnp.float32),
        ],
    )(inputs, hidden, W1, b1, W_ihT, b_ih, W_hhT, b_hh, Wl, Wr)


def _sc_gather_body(xl_hbm, xr_hbm, src_hbm, dst_hbm,
                    s_out,
                    sidx, didx, buf, sem1, sem2):
    cid = lax.axis_index("c")
    sid = lax.axis_index("s")
    wid = sid * NC + cid

    def body(i, carry):
        ch = wid + i * NW

        @pl.when(ch < NCHUNK)
        def _():
            base = ch * CHUNK
            pltpu.sync_copy(src_hbm.at[pl.ds(base, CHUNK)], sidx)
            pltpu.sync_copy(dst_hbm.at[pl.ds(base, CHUNK)], didx)
            pltpu.async_copy(xl_hbm.at[sidx], buf, sem1).wait()
            pltpu.async_copy(xr_hbm.at[didx], buf, sem2, add=True).wait()
            pltpu.sync_copy(buf, s_out.at[pl.ds(base, CHUNK)])

        return carry

    lax.fori_loop(0, LOOPS, body, 0)


_sc_gather = functools.partial(
    pl.kernel,
    out_type=jax.ShapeDtypeStruct((E, 2 * H), jnp.float32),
    mesh=plsc.VectorSubcoreMesh(core_axis_name="c", subcore_axis_name="s",
                                num_cores=NC, num_subcores=NS),
    scratch_types=[
        pltpu.VMEM((CHUNK,), jnp.int32),
        pltpu.VMEM((CHUNK,), jnp.int32),
        pltpu.VMEM((CHUNK, 2 * H), jnp.float32),
        pltpu.SemaphoreType.DMA,
        pltpu.SemaphoreType.DMA,
    ],
)(_sc_gather_body)


def _edge_body(s_ref, ea_ref, We_ref, att_ref, ex_ref, w_ref):
    xls = s_ref[:, :H]
    e = jnp.dot(ea_ref[...], We_ref[...], preferred_element_type=jnp.float32)
    m = xls + s_ref[:, H:] + e
    lr = jnp.where(m > 0, m, 0.2 * m)
    t = jnp.dot(lr, att_ref[...], preferred_element_type=jnp.float32)
    ex = jnp.exp(t)
    ex_ref[...] = ex
    w_ref[...] = ex * xls


def _edge_dense(s, edge_attr, We, att):
    nblk = E // EDGE_BLK
    full = lambda shape: pl.BlockSpec(shape, lambda i: (0,) * len(shape))
    row = lambda c: pl.BlockSpec((EDGE_BLK, c), lambda i: (i, 0))
    return pl.pallas_call(
        _edge_body,
        grid=(nblk,),
        in_specs=[row(2 * H), row(EDGE_DIM),
                  full((EDGE_DIM, H)), full((H, 1))],
        out_specs=[row(1), row(H)],
        out_shape=[
            jax.ShapeDtypeStruct((E, 1), jnp.float32),
            jax.ShapeDtypeStruct((E, H), jnp.float32),
        ],
    )(s, edge_attr, We, att)


def _final_body(num_ref, den_ref, bias_ref, W2_ref, b2_ref, q_ref):
    out = num_ref[...] / (den_ref[...] + 1e-16) + bias_ref[...]
    h = jax.nn.relu(out)
    q_ref[...] = jnp.dot(h, W2_ref[...], preferred_element_type=jnp.float32) + b2_ref[...]


def _final(num, den, gnn_bias, W2, b2):
    nblk = N // ROW_BLK
    full = lambda shape: pl.BlockSpec(shape, lambda i: (0,) * len(shape))
    row = lambda c: pl.BlockSpec((ROW_BLK, c), lambda i: (i, 0))
    return pl.pallas_call(
        _final_body,
        grid=(nblk,),
        in_specs=[row(H), row(1), full((1, H)), full((H, N_ACT)), full((1, N_ACT))],
        out_specs=row(N_ACT),
        out_shape=jax.ShapeDtypeStruct((N, N_ACT), jnp.float32),
    )(num, den, gnn_bias, W2, b2)


def kernel(inputs, hidden_states, edge_index, edge_attr,
           W1, b1, W_ih, b_ih, W_hh, b_hh,
           Wl, Wr, We, att, gnn_bias, W2, b2):
    h_s, xl128, xr128 = _dense_pre(
        inputs, hidden_states.reshape(-1, HH),
        W1, b1.reshape(1, H),
        W_ih.T, b_ih.reshape(1, 3 * HH),
        W_hh.T, b_hh.reshape(1, 3 * HH),
        Wl, Wr)

    src = edge_index[0]
    dst = edge_index[1]
    s = _sc_gather(xl128, xr128, src, dst)
    ex, w = _edge_dense(s, edge_attr, We, att.reshape(H, 1))

    den = jax.ops.segment_sum(ex[:, 0], dst, num_segments=N)
    num = jax.ops.segment_sum(w, dst, num_segments=N)

    q = _final(num, den.reshape(N, 1), gnn_bias.reshape(1, H),
               W2, b2.reshape(1, N_ACT))
    return (q, h_s)


# speedup vs baseline: 4.9246x; 1.2104x over previous
import functools

import jax
import jax.numpy as jnp
from jax import lax
from jax.experimental import pallas as pl
from jax.experimental.pallas import tpu as pltpu
from jax.experimental.pallas import tpu_sc as plsc

N = 10000
E = 320000
D_IN = 128
H = 64
HH = 2 * H
N_ACT = 10
EDGE_DIM = 5

ROW_BLK = 1000
EDGE_BLK = 8000

NC = 2
NS = 16
NW = NC * NS
CHUNK = 128
NCHUNK = E // CHUNK
LOOPS = -(-NCHUNK // NW)


def _dense_pre_body(inp_ref, hid_ref, W1_ref, b1_ref, WihT_ref, bih_ref,
                    WhhT_ref, bhh_ref, Wl_ref, Wr_ref,
                    hs_ref, xl128_ref, xr128_ref):
    x = jnp.dot(inp_ref[...], W1_ref[...], preferred_element_type=jnp.float32)
    x = jax.nn.relu(x + b1_ref[...])
    h = hid_ref[...]
    gi = jnp.dot(x, WihT_ref[...], preferred_element_type=jnp.float32) + bih_ref[...]
    gh = jnp.dot(h, WhhT_ref[...], preferred_element_type=jnp.float32) + bhh_ref[...]
    r = jax.nn.sigmoid(gi[:, :HH] + gh[:, :HH])
    z = jax.nn.sigmoid(gi[:, HH:2 * HH] + gh[:, HH:2 * HH])
    n = jnp.tanh(gi[:, 2 * HH:] + r * gh[:, 2 * HH:])
    h_s = (1.0 - z) * n + z * h
    hs_ref[...] = h_s
    xl = jnp.dot(h_s, Wl_ref[...], preferred_element_type=jnp.float32)
    xr = jnp.dot(h_s, Wr_ref[...], preferred_element_type=jnp.float32)
    zero = jnp.zeros_like(xl)
    xl128_ref[...] = jnp.concatenate([xl, zero], axis=1)
    xr128_ref[...] = jnp.concatenate([zero, xr], axis=1)


def _dense_pre(inputs, hidden, W1, b1, W_ihT, b_ih, W_hhT, b_hh, Wl, Wr):
    nblk = N // ROW_BLK
    full = lambda shape: pl.BlockSpec(shape, lambda i: (0,) * len(shape))
    row = lambda c: pl.BlockSpec((ROW_BLK, c), lambda i: (i, 0))
    return pl.pallas_call(
        _dense_pre_body,
        grid=(nblk,),
        in_specs=[
            row(D_IN), row(HH),
            full((D_IN, H)), full((1, H)),
            full((H, 3 * HH)), full((1, 3 * HH)),
            full((HH, 3 * HH)), full((1, 3 * HH)),
            full((HH, H)), full((HH, H)),
        ],
        out_specs=[row(HH), row(2 * H), row(2 * H)],
        out_shape=[
            jax.ShapeDtypeStruct((N, HH), jnp.float32),
            jax.ShapeDtypeStruct((N, 2 * H), jnp.float32),
            jax.ShapeDtypeStruct((N, 2 * H), jnp.float32),
        ],
    )(inputs, hidden, W1, b1, W_ihT, b_ih, W_hhT, b_hh, Wl, Wr)


def _sc_gather_body(xl_hbm, xr_hbm, src_hbm, dst_hbm,
                    s_out,
                    sidx, didx, buf, sem1, sem2):
    cid = lax.axis_index("c")
    sid = lax.axis_index("s")
    wid = sid * NC + cid

    def body(i, carry):
        ch = wid + i * NW

        @pl.when(ch < NCHUNK)
        def _():
            base = ch * CHUNK
            pltpu.sync_copy(src_hbm.at[pl.ds(base, CHUNK)], sidx)
            pltpu.sync_copy(dst_hbm.at[pl.ds(base, CHUNK)], didx)
            pltpu.async_copy(xl_hbm.at[sidx], buf, sem1).wait()
            pltpu.async_copy(xr_hbm.at[didx], buf, sem2, add=True).wait()
            pltpu.sync_copy(buf, s_out.at[pl.ds(base, CHUNK)])

        return carry

    lax.fori_loop(0, LOOPS, body, 0)


_sc_gather = functools.partial(
    pl.kernel,
    out_type=jax.ShapeDtypeStruct((E, 2 * H), jnp.float32),
    mesh=plsc.VectorSubcoreMesh(core_axis_name="c", subcore_axis_name="s",
                                num_cores=NC, num_subcores=NS),
    scratch_types=[
        pltpu.VMEM((CHUNK,), jnp.int32),
        pltpu.VMEM((CHUNK,), jnp.int32),
        pltpu.VMEM((CHUNK, 2 * H), jnp.float32),
        pltpu.SemaphoreType.DMA,
        pltpu.SemaphoreType.DMA,
    ],
)(_sc_gather_body)


def _edge_body(s_ref, ea_ref, We_ref, att_ref, w_ref):
    xls = s_ref[:, :H]
    e = jnp.dot(ea_ref[...], We_ref[...], preferred_element_type=jnp.float32)
    m = xls + s_ref[:, H:] + e
    lr = jnp.where(m > 0, m, 0.2 * m)
    t = jnp.dot(lr, att_ref[...], preferred_element_type=jnp.float32)
    ex = jnp.exp(t)
    w_ref[...] = jnp.concatenate([ex * xls, ex], axis=1)


def _edge_dense(s, edge_attr, We, att):
    nblk = E // EDGE_BLK
    full = lambda shape: pl.BlockSpec(shape, lambda i: (0,) * len(shape))
    row = lambda c: pl.BlockSpec((EDGE_BLK, c), lambda i: (i, 0))
    return pl.pallas_call(
        _edge_body,
        grid=(nblk,),
        in_specs=[row(2 * H), row(EDGE_DIM),
                  full((EDGE_DIM, H)), full((H, 1))],
        out_specs=row(H + 1),
        out_shape=jax.ShapeDtypeStruct((E, H + 1), jnp.float32),
    )(s, edge_attr, We, att)


def _final_body(seg_ref, bias_ref, W2_ref, b2_ref, q_ref):
    num = seg_ref[:, :H]
    den = seg_ref[:, H:]
    out = num / (den + 1e-16) + bias_ref[...]
    h = jax.nn.relu(out)
    q_ref[...] = jnp.dot(h, W2_ref[...], preferred_element_type=jnp.float32) + b2_ref[...]


def _final(seg, gnn_bias, W2, b2):
    nblk = N // ROW_BLK
    full = lambda shape: pl.BlockSpec(shape, lambda i: (0,) * len(shape))
    row = lambda c: pl.BlockSpec((ROW_BLK, c), lambda i: (i, 0))
    return pl.pallas_call(
        _final_body,
        grid=(nblk,),
        in_specs=[row(H + 1), full((1, H)), full((H, N_ACT)), full((1, N_ACT))],
        out_specs=row(N_ACT),
        out_shape=jax.ShapeDtypeStruct((N, N_ACT), jnp.float32),
    )(seg, gnn_bias, W2, b2)


def kernel(inputs, hidden_states, edge_index, edge_attr,
           W1, b1, W_ih, b_ih, W_hh, b_hh,
           Wl, Wr, We, att, gnn_bias, W2, b2):
    h_s, xl128, xr128 = _dense_pre(
        inputs, hidden_states.reshape(-1, HH),
        W1, b1.reshape(1, H),
        W_ih.T, b_ih.reshape(1, 3 * HH),
        W_hh.T, b_hh.reshape(1, 3 * HH),
        Wl, Wr)

    src = edge_index[0]
    dst = edge_index[1]
    s = _sc_gather(xl128, xr128, src, dst)
    w = _edge_dense(s, edge_attr, We, att.reshape(H, 1))

    seg = jax.ops.segment_sum(w, dst, num_segments=N)

    q = _final(seg, gnn_bias.reshape(1, H),
               W2, b2.reshape(1, N_ACT))
    return (q, h_s)
